# Initial kernel scaffold; baseline (speedup 1.0000x reference)
import jax
import jax.numpy as jnp
from jax.experimental import pallas as pl


def kernel(user_weight, item_weight, adj_indices, adj_values):
    raise NotImplementedError("write your pallas kernel here")


# speedup vs baseline: 5.6525x; 5.6525x over previous
import functools

import jax
import jax.numpy as jnp
from jax import lax
from jax.experimental import pallas as pl
from jax.experimental.pallas import tpu as pltpu
from jax.experimental.pallas import tpu_sc as plsc

_L = 16
_NC = 2
_NS = 16
_CHUNK = 128
_BLK = 8
_N_LAYERS = 3


def _build_sc_call(n, h, n_chunks_tile, rc, nrc):
    n_blocks_tile = n_chunks_tile // _BLK
    rows_tile = rc * nrc
    ngrp = h // _L

    mesh = plsc.VectorSubcoreMesh(core_axis_name="c", subcore_axis_name="s")
    out_sds = jax.ShapeDtypeStruct((_NC * n, h), jnp.float32)

    @functools.partial(
        pl.kernel,
        out_type=[out_sds] * 4,
        mesh=mesh,
        compiler_params=pltpu.CompilerParams(use_tc_tiling_on_sc=False),
        scratch_types=[
            pltpu.VMEM((_BLK, _CHUNK), jnp.int32),
            pltpu.VMEM((_BLK, _CHUNK), jnp.int32),
            pltpu.VMEM((_BLK, _CHUNK), jnp.float32),
            pltpu.VMEM((_CHUNK, h), jnp.float32),
            pltpu.VMEM((rc, h), jnp.float32),
            pltpu.VMEM((rc, h), jnp.float32),
            pltpu.VMEM_SHARED((n, h), jnp.float32),
        ],
    )
    def sc_call(x0, rows_b, cols_b, vals_b, mean_o, x1_o, x2_o, x3_o,
                idx_r, idx_c, valb, gath, zbuf, dbuf, acc):
        c = lax.axis_index("c")
        s = lax.axis_index("s")
        row_base = s * rows_tile
        hbm_base = c * n + row_base

        def _z(r, carry):
            for g in range(ngrp):
                zbuf[r, pl.ds(g * _L, _L)] = jnp.zeros((_L,), jnp.float32)
            return carry
        lax.fori_loop(0, rc, _z, None)

        def layer(src, dst):
            def _zero(k, carry):
                pltpu.sync_copy(zbuf, acc.at[pl.ds(row_base + k * rc, rc)])
                return carry
            lax.fori_loop(0, nrc, _zero, None)
            plsc.subcore_barrier()

            def _block(b, carry):
                base = (s * n_blocks_tile + b) * _BLK
                pltpu.sync_copy(rows_b.at[pl.ds(base, _BLK)], idx_r)
                pltpu.sync_copy(cols_b.at[c, pl.ds(base, _BLK)], idx_c)
                pltpu.sync_copy(vals_b.at[pl.ds(base, _BLK)], valb)
                for j in range(_BLK):
                    pltpu.sync_copy(src.at[idx_c.at[j]], gath)

                    def _mul(e16, carry2):
                        base_e = e16 * _L
                        val16 = valb[j, pl.ds(base_e, _L)]
                        for l in range(_L):
                            vv = jnp.full((_L,), val16[l], jnp.float32)
                            for g in range(ngrp):
                                sl = pl.ds(g * _L, _L)
                                gath[base_e + l, sl] = gath[base_e + l, sl] * vv
                        return carry2
                    lax.fori_loop(0, _CHUNK // _L, _mul, None)
                    pltpu.sync_copy(gath, acc.at[idx_r.at[j]], add=True)
                return carry
            lax.fori_loop(0, n_blocks_tile, _block, None)
            plsc.subcore_barrier()

            def _drain(k, carry):
                pltpu.sync_copy(acc.at[pl.ds(row_base + k * rc, rc)], dbuf)
                pltpu.sync_copy(dbuf, dst.at[pl.ds(hbm_base + k * rc, rc)])
                return carry
            lax.fori_loop(0, nrc, _drain, None)

        layer(x0, x1_o)
        layer(x1_o, x2_o)
        layer(x2_o, x3_o)
        plsc.subcore_barrier()

        def _mean(k, carry):
            sl = pl.ds(hbm_base + k * rc, rc)
            pltpu.sync_copy(x0.at[sl], dbuf)
            for i, xsrc in enumerate((x1_o, x2_o, x3_o)):
                pltpu.sync_copy(xsrc.at[sl], zbuf)
                scale = 0.25 if i == 2 else 1.0

                def _acc(r, carry2):
                    for g in range(ngrp):
                        ssl = pl.ds(g * _L, _L)
                        dbuf[r, ssl] = (dbuf[r, ssl] + zbuf[r, ssl]) * scale
                    return carry2
                lax.fori_loop(0, rc, _acc, None)
            pltpu.sync_copy(dbuf, mean_o.at[sl])
            return carry
        lax.fori_loop(0, nrc, _mean, None)

    return sc_call


def kernel(user_weight, item_weight, adj_indices, adj_values):
    n_users, d = user_weight.shape
    n_items = item_weight.shape[0]
    n = n_users + n_items
    h = d // 2
    e = adj_values.shape[0]

    per_tile = -(-e // (_NS * _BLK * _CHUNK)) * (_BLK * _CHUNK)
    e_pad = per_tile * _NS
    pad = e_pad - e
    rows = jnp.pad(adj_indices[0], (0, pad))
    cols = jnp.pad(adj_indices[1], (0, pad))
    vals = jnp.pad(adj_values, (0, pad))

    rows_tile = n // _NS
    rc = 1
    for cand in range(2, 129):
        if rows_tile % cand == 0:
            rc = cand
    nrc = rows_tile // rc

    all_emb = jnp.concatenate([user_weight, item_weight], axis=0)
    x0 = jnp.concatenate([all_emb[:, :h], all_emb[:, h:]], axis=0)

    rows_b = rows.reshape(-1, _CHUNK)
    cols_b = jnp.stack([cols, cols + n]).reshape(2, -1, _CHUNK)
    vals_b = vals.reshape(-1, _CHUNK)

    sc_call = _build_sc_call(n, h, e_pad // (_NS * _CHUNK), rc, nrc)
    mean_flat, _, _, _ = sc_call(x0, rows_b, cols_b, vals_b)

    out = jnp.concatenate([mean_flat[:n], mean_flat[n:]], axis=1)
    return out[:n_users], out[n_users:]


# speedup vs baseline: 8.0744x; 1.4285x over previous
import functools

import jax
import jax.numpy as jnp
from jax import lax
from jax.experimental import pallas as pl
from jax.experimental.pallas import tpu as pltpu
from jax.experimental.pallas import tpu_sc as plsc

_L = 16
_NC = 2
_NS = 16
_CHUNK = 128
_BLK = 8
_N_LAYERS = 3


def _build_sc_call(n, h, n_chunks_tile, rc, nrc):
    n_blocks_tile = n_chunks_tile // _BLK
    rows_tile = rc * nrc
    ngrp = h // _L

    mesh = plsc.VectorSubcoreMesh(core_axis_name="c", subcore_axis_name="s")
    out_sds = jax.ShapeDtypeStruct((_NC * n, h), jnp.float32)

    @functools.partial(
        pl.kernel,
        out_type=[out_sds] * 4,
        mesh=mesh,
        compiler_params=pltpu.CompilerParams(use_tc_tiling_on_sc=False),
        scratch_types=[
            pltpu.VMEM((_BLK, _CHUNK), jnp.int32),
            pltpu.VMEM((_BLK, _CHUNK), jnp.int32),
            pltpu.VMEM((_BLK, _CHUNK), jnp.float32),
            pltpu.VMEM((_CHUNK, h), jnp.float32),
            pltpu.VMEM((_CHUNK, h), jnp.float32),
            pltpu.VMEM((rc, h), jnp.float32),
            pltpu.VMEM((rc, h), jnp.float32),
            pltpu.VMEM_SHARED((n, h), jnp.float32),
            pltpu.SemaphoreType.DMA,
            pltpu.SemaphoreType.DMA,
        ],
    )
    def sc_call(x0, rows_b, cols_b, vals_b, mean_o, x1_o, x2_o, x3_o,
                idx_r, idx_c, valb, gath0, gath1, zbuf, dbuf, acc,
                sem0, sem1):
        gaths = (gath0, gath1)
        sems = (sem0, sem1)
        c = lax.axis_index("c")
        s = lax.axis_index("s")
        row_base = s * rows_tile
        hbm_base = c * n + row_base

        def _z(r, carry):
            for g in range(ngrp):
                zbuf[r, pl.ds(g * _L, _L)] = jnp.zeros((_L,), jnp.float32)
            return carry
        lax.fori_loop(0, rc, _z, None)

        def layer(src, dst):
            def _zero(k, carry):
                pltpu.sync_copy(zbuf, acc.at[pl.ds(row_base + k * rc, rc)])
                return carry
            lax.fori_loop(0, nrc, _zero, None)
            plsc.subcore_barrier()

            def _block(b, carry):
                base = (s * n_blocks_tile + b) * _BLK
                pltpu.sync_copy(rows_b.at[pl.ds(base, _BLK)], idx_r)
                pltpu.sync_copy(cols_b.at[c, pl.ds(base, _BLK)], idx_c)
                pltpu.sync_copy(vals_b.at[pl.ds(base, _BLK)], valb)
                d_next = pltpu.async_copy(src.at[idx_c.at[0]], gaths[0],
                                          sems[0])
                for j in range(_BLK):
                    gath = gaths[j % 2]
                    d_cur = d_next
                    if j + 1 < _BLK:
                        d_next = pltpu.async_copy(
                            src.at[idx_c.at[j + 1]], gaths[(j + 1) % 2],
                            sems[(j + 1) % 2])
                    d_cur.wait()

                    @plsc.parallel_loop(0, _CHUNK // _L)
                    def _mul(e16):
                        base_e = e16 * _L
                        val16 = valb[j, pl.ds(base_e, _L)]
                        for l in range(_L):
                            vv = jnp.full((_L,), val16[l], jnp.float32)
                            for g in range(ngrp):
                                sl = pl.ds(g * _L, _L)
                                gath[base_e + l, sl] = gath[base_e + l, sl] * vv
                    pltpu.sync_copy(gath, acc.at[idx_r.at[j]], add=True)
                return carry
            lax.fori_loop(0, n_blocks_tile, _block, None)
            plsc.subcore_barrier()

            def _drain(k, carry):
                pltpu.sync_copy(acc.at[pl.ds(row_base + k * rc, rc)], dbuf)
                pltpu.sync_copy(dbuf, dst.at[pl.ds(hbm_base + k * rc, rc)])
                return carry
            lax.fori_loop(0, nrc, _drain, None)

        layer(x0, x1_o)
        layer(x1_o, x2_o)
        layer(x2_o, x3_o)
        plsc.subcore_barrier()

        def _mean(k, carry):
            sl = pl.ds(hbm_base + k * rc, rc)
            pltpu.sync_copy(x0.at[sl], dbuf)
            for i, xsrc in enumerate((x1_o, x2_o, x3_o)):
                pltpu.sync_copy(xsrc.at[sl], zbuf)
                scale = 0.25 if i == 2 else 1.0

                def _acc(r, carry2):
                    for g in range(ngrp):
                        ssl = pl.ds(g * _L, _L)
                        dbuf[r, ssl] = (dbuf[r, ssl] + zbuf[r, ssl]) * scale
                    return carry2
                lax.fori_loop(0, rc, _acc, None)
            pltpu.sync_copy(dbuf, mean_o.at[sl])
            return carry
        lax.fori_loop(0, nrc, _mean, None)

    return sc_call


def kernel(user_weight, item_weight, adj_indices, adj_values):
    n_users, d = user_weight.shape
    n_items = item_weight.shape[0]
    n = n_users + n_items
    h = d // 2
    e = adj_values.shape[0]

    per_tile = -(-e // (_NS * _BLK * _CHUNK)) * (_BLK * _CHUNK)
    e_pad = per_tile * _NS
    pad = e_pad - e
    rows = jnp.pad(adj_indices[0], (0, pad))
    cols = jnp.pad(adj_indices[1], (0, pad))
    vals = jnp.pad(adj_values, (0, pad))

    rows_tile = n // _NS
    rc = 1
    for cand in range(2, 129):
        if rows_tile % cand == 0:
            rc = cand
    nrc = rows_tile // rc

    all_emb = jnp.concatenate([user_weight, item_weight], axis=0)
    x0 = jnp.concatenate([all_emb[:, :h], all_emb[:, h:]], axis=0)

    rows_b = rows.reshape(-1, _CHUNK)
    cols_b = jnp.stack([cols, cols + n]).reshape(2, -1, _CHUNK)
    vals_b = vals.reshape(-1, _CHUNK)

    sc_call = _build_sc_call(n, h, e_pad // (_NS * _CHUNK), rc, nrc)
    mean_flat, _, _, _ = sc_call(x0, rows_b, cols_b, vals_b)

    out = jnp.concatenate([mean_flat[:n], mean_flat[n:]], axis=1)
    return out[:n_users], out[n_users:]
